# baseline (speedup 1.0000x reference)
import functools

import jax
import jax.numpy as jnp
from jax import lax
from jax.experimental import pallas as pl
from jax.experimental.pallas import tpu as pltpu
from jax.experimental.pallas import tpu_sc as plsc

_N_NODES = 10000
_N_EDGES = 320000
_N_GRAPHS = 64
_NW = 32
_NPW = 320
_NPAD = _NW * _NPW
_WLREG = _N_EDGES + 64

_MESH = plsc.VectorSubcoreMesh(
    core_axis_name="c", subcore_axis_name="s", num_cores=2, num_subcores=16)


def _wid():
    return lax.axis_index("s") * 2 + lax.axis_index("c")


def _proj_body(x_ref, wd_ref, ws_ref, ba_ref, p_ref, q_ref):
    x = x_ref[...]
    p_ref[...] = (jnp.dot(x, wd_ref[...], preferred_element_type=jnp.float32)
                  + ba_ref[...])
    q_ref[...] = jnp.dot(x, ws_ref[...], preferred_element_type=jnp.float32)


def _project(x, wd, ws, ba):
    n, hin = x.shape
    h = wd.shape[1]
    bm = 2000
    return pl.pallas_call(
        _proj_body,
        grid=(n // bm,),
        in_specs=[
            pl.BlockSpec((bm, hin), lambda i: (i, 0)),
            pl.BlockSpec((hin, h), lambda i: (0, 0)),
            pl.BlockSpec((hin, h), lambda i: (0, 0)),
            pl.BlockSpec((1, h), lambda i: (0, 0)),
        ],
        out_specs=[
            pl.BlockSpec((bm, h), lambda i: (i, 0)),
            pl.BlockSpec((bm, h), lambda i: (i, 0)),
        ],
        out_shape=[
            jax.ShapeDtypeStruct((n, h), jnp.float32),
            jax.ShapeDtypeStruct((n, h), jnp.float32),
        ],
    )(x, wd, ws, ba.reshape(1, h))


def _edge_mlp_body(a_ref, b_ref, wb_ref, c_ref):
    h = jnp.maximum(a_ref[...] + b_ref[...], 0.0)
    c_ref[...] = jnp.dot(h, wb_ref[...], preferred_element_type=jnp.float32)


def _edge_mlp(pa, pb, wb):
    e, h = pa.shape
    bm = 2000
    return pl.pallas_call(
        _edge_mlp_body,
        grid=(e // bm,),
        in_specs=[
            pl.BlockSpec((bm, h), lambda i: (i, 0)),
            pl.BlockSpec((bm, h), lambda i: (i, 0)),
            pl.BlockSpec((h, h), lambda i: (0, 0)),
        ],
        out_specs=pl.BlockSpec((bm, h), lambda i: (i, 0)),
        out_shape=jax.ShapeDtypeStruct((e, h), jnp.float32),
    )(pa, pb, wb)


def _pool_body(x_ref, batch_ref, wfc_ref, bfc_ref, out_ref, pooled_ref):
    xb = x_ref[...]
    bvec = batch_ref[...]

    def body(g, c):
        mg = jnp.max(jnp.where(bvec == g, xb, -jnp.inf), axis=0,
                     keepdims=True)
        pooled_ref[pl.ds(g, 1), :] = jnp.where(mg == -jnp.inf, 0.0, mg)
        return c

    lax.fori_loop(0, _N_GRAPHS, body, 0)
    out_ref[...] = (jnp.dot(pooled_ref[...], wfc_ref[...],
                            preferred_element_type=jnp.float32) + bfc_ref[...])


def _pool(x3, batch, wfc, bfc):
    n, h = x3.shape
    hout = wfc.shape[1]
    return pl.pallas_call(
        _pool_body,
        out_shape=jax.ShapeDtypeStruct((_N_GRAPHS, hout), jnp.float32),
        scratch_shapes=[pltpu.VMEM((_N_GRAPHS, h), jnp.float32)],
    )(x3, batch.reshape(n, 1), wfc, bfc.reshape(1, hout))


@functools.cache
def _make_gather(h):
    ch = 128
    epw = _N_EDGES // _NW
    nfull = epw // ch
    rem = epw - nfull * ch

    @functools.partial(
        pl.kernel,
        out_type=[
            jax.ShapeDtypeStruct((_N_EDGES, h), jnp.float32),
            jax.ShapeDtypeStruct((_N_EDGES, h), jnp.float32),
        ],
        mesh=_MESH,
        scratch_types=[
            pltpu.VMEM((ch,), jnp.int32),
            pltpu.VMEM((ch,), jnp.int32),
            pltpu.VMEM((ch, h), jnp.float32),
            pltpu.VMEM((ch, h), jnp.float32),
            pltpu.SemaphoreType.DMA,
            pltpu.SemaphoreType.DMA,
        ],
    )
    def k(p_hbm, q_hbm, dst_hbm, src_hbm, outa, outb,
          idx_d, idx_s, buf_a, buf_b, sem_a, sem_b):
        base = _wid() * epw

        def do_chunk(off, n):
            pltpu.sync_copy(dst_hbm.at[pl.ds(off, n)], idx_d.at[pl.ds(0, n)])
            pltpu.sync_copy(src_hbm.at[pl.ds(off, n)], idx_s.at[pl.ds(0, n)])
            cpa = pltpu.async_copy(p_hbm.at[idx_d.at[pl.ds(0, n)]],
                                   buf_a.at[pl.ds(0, n)], sem_a)
            cpb = pltpu.async_copy(q_hbm.at[idx_s.at[pl.ds(0, n)]],
                                   buf_b.at[pl.ds(0, n)], sem_b)
            cpa.wait()
            cpb.wait()
            pltpu.sync_copy(buf_a.at[pl.ds(0, n)], outa.at[pl.ds(off, n)])
            pltpu.sync_copy(buf_b.at[pl.ds(0, n)], outb.at[pl.ds(off, n)])

        def body(i, carry):
            do_chunk(base + i * ch, ch)
            return carry

        lax.fori_loop(0, nfull, body, 0)
        do_chunk(base + nfull * ch, rem)

    return k


def _fire_drain_rmw(cflat, sem, wl, wlbase, rows, table, nthis, rb, h, nk):
    def fire(j, c):
        p = wl[pl.ds((wlbase + j) * 16, 16)][0]
        e = jnp.where(j < nthis, p >> 9, 0)
        pltpu.async_copy(cflat.at[pl.ds(e * h, h)],
                         rows.at[pl.ds(j * h, h)], sem)
        return c

    lax.fori_loop(0, rb, fire, 0)
    pltpu.make_async_copy(cflat.at[pl.ds(0, rb * h)], rows, sem).wait()

    def rmw(j, c):
        p = wl[pl.ds((wlbase + j) * 16, 16)][0]
        n_i = jnp.where(j < nthis, p & 511, _NPW)
        for kk in range(nk):
            t = table[n_i, pl.ds(kk * 16, 16)]
            r = rows[pl.ds(j * h + kk * 16, 16)]
            table[n_i, pl.ds(kk * 16, 16)] = jnp.maximum(t, r)
        return c

    lax.fori_loop(0, rb, rmw, 0)


def _table_init(table, nk):
    neg_inf = jnp.full((16,), -jnp.inf, jnp.float32)

    def init_row(j, c):
        for kk in range(nk):
            table[j, pl.ds(kk * 16, 16)] = neg_inf
        return c

    lax.fori_loop(0, _NPW + 1, init_row, 0)


def _table_finalize(table, bb_v, out_hbm, lo, nk):
    neg_inf = jnp.full((16,), -jnp.inf, jnp.float32)

    def fin_row(j, c):
        for kk in range(nk):
            t = table[j, pl.ds(kk * 16, 16)]
            b = bb_v[pl.ds(kk * 16, 16)]
            table[j, pl.ds(kk * 16, 16)] = jnp.where(t == neg_inf, 0.0, t + b)
        return c

    lax.fori_loop(0, _NPW, fin_row, 0)
    pltpu.sync_copy(table.at[pl.ds(0, _NPW)], out_hbm.at[pl.ds(lo, _NPW)])


@functools.cache
def _make_segmax_build(h):
    ch2 = 256
    ngrp = ch2 // 16
    wlcap = 768
    flush_at = 512
    rb = 64
    nk = h // 16

    @functools.partial(
        pl.kernel,
        out_type=[
            jax.ShapeDtypeStruct((_NPAD, h), jnp.float32),
            jax.ShapeDtypeStruct((_NW * _WLREG * 16,), jnp.int32),
            jax.ShapeDtypeStruct((_NW * 16,), jnp.int32),
        ],
        mesh=_MESH,
        scratch_types=[
            pltpu.VMEM((_NPW + 1, h), jnp.float32),
            pltpu.VMEM((ch2,), jnp.int32),
            pltpu.VMEM(((wlcap + 1) * 16,), jnp.int32),
            pltpu.VMEM((rb * h,), jnp.float32),
            pltpu.VMEM((h,), jnp.float32),
            pltpu.VMEM((16,), jnp.int32),
            pltpu.SemaphoreType.DMA,
            pltpu.SemaphoreType.DMA,
        ],
    )
    def k(cflat, dst_hbm, bb_hbm, out_hbm, wl_hbm, wcnt_hbm,
          table, dvec, wl, rows, bb_v, stg, sem, sem2):
        w = _wid()
        lo = w * _NPW
        zeros_i = jnp.zeros((16,), jnp.int32)

        pltpu.sync_copy(bb_hbm, bb_v)
        _table_init(table, nk)

        def flush(cnt, cursor):
            nsub = (cnt + rb - 1) // rb

            def sub(s, c):
                base = s * rb
                nthis = cnt - base
                pltpu.async_copy(
                    wl.at[pl.ds(base * 16, rb * 16)],
                    wl_hbm.at[pl.ds((w * _WLREG + cursor + base) * 16,
                                    rb * 16)],
                    sem2)
                _fire_drain_rmw(cflat, sem, wl, base, rows, table, nthis,
                                rb, h, nk)
                pltpu.make_async_copy(
                    cflat.at[pl.ds(0, rb * 16)],
                    wl.at[pl.ds(0, rb * 16)], sem2).wait()
                return c

            lax.fori_loop(0, nsub, sub, 0)
            return cursor + cnt

        def chunk(ci, carry):
            cnt0, cursor0 = carry
            coff = ci * ch2
            pltpu.sync_copy(dst_hbm.at[pl.ds(coff, ch2)], dvec)

            def grp(g, cnt_in):
                d = dvec[pl.ds(g * 16, 16)]
                dl = d - lo
                cnt = cnt_in
                for lane in range(16):
                    dl_l = dl[lane]
                    ok = (dl_l >= 0) & (dl_l < _NPW)
                    e = coff + g * 16 + lane
                    val = zeros_i + (e * 512 + dl_l)
                    slot = jnp.where(ok, cnt, wlcap)
                    wl[pl.ds(slot * 16, 16)] = val
                    cnt = cnt + ok.astype(jnp.int32)
                return cnt

            cnt1 = lax.fori_loop(0, ngrp, grp, cnt0)
            return lax.cond(cnt1 >= flush_at,
                            lambda a: (0, flush(a[0], a[1])),
                            lambda a: a, (cnt1, cursor0))

        cnt_f, cursor_f = lax.fori_loop(0, _N_EDGES // ch2, chunk, (0, 0))
        cursor_f = lax.cond(cnt_f > 0,
                            lambda a: flush(a[0], a[1]),
                            lambda a: a[1], (cnt_f, cursor_f))

        stg[pl.ds(0, 16)] = zeros_i + cursor_f
        pltpu.sync_copy(stg, wcnt_hbm.at[pl.ds(w * 16, 16)])

        _table_finalize(table, bb_v, out_hbm, lo, nk)

    return k


@functools.cache
def _make_segmax_apply(h):
    rb = 64
    nk = h // 16

    @functools.partial(
        pl.kernel,
        out_type=jax.ShapeDtypeStruct((_NPAD, h), jnp.float32),
        mesh=_MESH,
        scratch_types=[
            pltpu.VMEM((_NPW + 1, h), jnp.float32),
            pltpu.VMEM((rb * 16,), jnp.int32),
            pltpu.VMEM((rb * 16,), jnp.int32),
            pltpu.VMEM((rb * h,), jnp.float32),
            pltpu.VMEM((rb * h,), jnp.float32),
            pltpu.VMEM((h,), jnp.float32),
            pltpu.VMEM((16,), jnp.int32),
            pltpu.SemaphoreType.DMA,
            pltpu.SemaphoreType.DMA,
        ],
    )
    def k(cflat, wl_hbm, wcnt_hbm, bb_hbm, out_hbm,
          table, wlbuf0, wlbuf1, rows0, rows1, bb_v, stg, sem0, sem1):
        w = _wid()
        lo = w * _NPW
        wlbufs = (wlbuf0, wlbuf1)
        rowss = (rows0, rows1)
        sems = (sem0, sem1)

        pltpu.sync_copy(bb_hbm, bb_v)
        _table_init(table, nk)

        pltpu.sync_copy(wcnt_hbm.at[pl.ds(w * 16, 16)], stg)
        n = stg[pl.ds(0, 16)][0]
        nsub2 = (n + 2 * rb - 1) // (2 * rb)

        def fire(wlbuf, rows, sem, nthis):
            def f(j, c):
                p = wlbuf[pl.ds(j * 16, 16)][0]
                e = jnp.where(j < nthis, p >> 9, 0)
                pltpu.async_copy(cflat.at[pl.ds(e * h, h)],
                                 rows.at[pl.ds(j * h, h)], sem)
                return c
            lax.fori_loop(0, rb, f, 0)

        def rmw(wlbuf, rows, nthis):
            def r(j, c):
                p = wlbuf[pl.ds(j * 16, 16)][0]
                n_i = jnp.where(j < nthis, p & 511, _NPW)
                for kk in range(nk):
                    t = table[n_i, pl.ds(kk * 16, 16)]
                    rr = rows[pl.ds(j * h + kk * 16, 16)]
                    table[n_i, pl.ds(kk * 16, 16)] = jnp.maximum(t, rr)
                return c
            lax.fori_loop(0, rb, r, 0)

        def sub2(s2, c):
            for b in range(2):
                base = (s2 * 2 + b) * rb
                pltpu.sync_copy(
                    wl_hbm.at[pl.ds((w * _WLREG + base) * 16, rb * 16)],
                    wlbufs[b])
                fire(wlbufs[b], rowss[b], sems[b], n - base)
            for b in range(2):
                base = (s2 * 2 + b) * rb
                pltpu.make_async_copy(cflat.at[pl.ds(0, rb * h)],
                                      rowss[b], sems[b]).wait()
                rmw(wlbufs[b], rowss[b], n - base)
            return c

        lax.fori_loop(0, nsub2, sub2, 0)

        _table_finalize(table, bb_v, out_hbm, lo, nk)

    return k


_RBUF = 32
_EPW = _N_EDGES // _NW
_RCAP = _EPW + _RBUF


@functools.cache
def _make_route():
    ch = 400
    nch = _EPW // ch

    @functools.partial(
        pl.kernel,
        out_type=[
            jax.ShapeDtypeStruct((_NW * _NW * _RCAP * 16,), jnp.int32),
            jax.ShapeDtypeStruct((_NW * _NW * 16,), jnp.int32),
        ],
        mesh=_MESH,
        scratch_types=[
            pltpu.VMEM((ch,), jnp.int32),
            pltpu.VMEM((_NW * _RBUF * 16,), jnp.int32),
            pltpu.VMEM((_NW * 16,), jnp.int32),
            pltpu.VMEM((16,), jnp.int32),
            pltpu.SemaphoreType.DMA,
        ],
    )
    def k(dst_hbm, routed, rcnt, dvec, stage, cnt, stg, sem):
        v = _wid()
        base_e = v * _EPW
        zeros_i = jnp.zeros((16,), jnp.int32)

        def zinit(j, c):
            cnt[pl.ds(j * 16, 16)] = zeros_i
            return c
        lax.fori_loop(0, _NW, zinit, 0)

        def zinit2(j, c):
            stage[pl.ds(j * 16, 16)] = zeros_i
            return c
        lax.fori_loop(0, _NW * _RBUF, zinit2, 0)

        def chunk(ci, c):
            coff = ci * ch
            pltpu.sync_copy(dst_hbm.at[pl.ds(base_e + coff, ch)], dvec)

            def grp(g, c2):
                d = dvec[pl.ds(g * 16, 16)]
                for lane in range(16):
                    d_l = d[lane]
                    ow = ((d_l >> 6) * 6554) >> 15
                    dl = d_l - ow * 320
                    e = base_e + coff + g * 16 + lane
                    tc = cnt[pl.ds(ow * 16, 16)][0]
                    stage[pl.ds((ow * _RBUF + (tc & (_RBUF - 1))) * 16,
                                16)] = zeros_i + (e * 512 + dl)
                    cnt[pl.ds(ow * 16, 16)] = zeros_i + (tc + 1)

                    @pl.when((tc & (_RBUF - 1)) == _RBUF - 1)
                    def _():
                        pltpu.sync_copy(
                            stage.at[pl.ds(ow * _RBUF * 16, _RBUF * 16)],
                            routed.at[pl.ds(
                                ((v * _NW + ow) * _RCAP + tc - (_RBUF - 1))
                                * 16, _RBUF * 16)])
                return c2

            lax.fori_loop(0, ch // 16, grp, 0)
            return c

        lax.fori_loop(0, nch, chunk, 0)

        def fin(ow, c):
            tc = cnt[pl.ds(ow * 16, 16)][0]

            @pl.when((tc & (_RBUF - 1)) != 0)
            def _():
                pltpu.sync_copy(
                    stage.at[pl.ds(ow * _RBUF * 16, _RBUF * 16)],
                    routed.at[pl.ds(
                        ((v * _NW + ow) * _RCAP + (tc & ~(_RBUF - 1))) * 16,
                        _RBUF * 16)])

            stg[pl.ds(0, 16)] = zeros_i + tc
            pltpu.sync_copy(stg, rcnt.at[pl.ds((v * _NW + ow) * 16, 16)])
            return c

        lax.fori_loop(0, _NW, fin, 0)

    return k


@functools.cache
def _make_segmax_merge(h):
    rb = 64
    nk = h // 16

    @functools.partial(
        pl.kernel,
        out_type=[
            jax.ShapeDtypeStruct((_NPAD, h), jnp.float32),
            jax.ShapeDtypeStruct((_NW * _WLREG * 16,), jnp.int32),
            jax.ShapeDtypeStruct((_NW * 16,), jnp.int32),
        ],
        mesh=_MESH,
        scratch_types=[
            pltpu.VMEM((_NPW + 1, h), jnp.float32),
            pltpu.VMEM((rb * 16,), jnp.int32),
            pltpu.VMEM((rb * 16,), jnp.int32),
            pltpu.VMEM((rb * h,), jnp.float32),
            pltpu.VMEM((rb * h,), jnp.float32),
            pltpu.VMEM((h,), jnp.float32),
            pltpu.VMEM((16,), jnp.int32),
            pltpu.SemaphoreType.DMA,
            pltpu.SemaphoreType.DMA,
            pltpu.SemaphoreType.DMA,
        ],
    )
    def k(cflat, routed, rcnt, bb_hbm, out_hbm, wl_hbm, wcnt_hbm,
          table, wlbuf0, wlbuf1, rows0, rows1, bb_v, stg, sem0, sem1, sem2):
        w = _wid()
        lo = w * _NPW
        zeros_i = jnp.zeros((16,), jnp.int32)
        wlbufs = (wlbuf0, wlbuf1)
        rowss = (rows0, rows1)
        sems = (sem0, sem1)

        pltpu.sync_copy(bb_hbm, bb_v)
        _table_init(table, nk)

        def fire(wlbuf, rows, sem, nthis):
            def f(j, c):
                p = wlbuf[pl.ds(j * 16, 16)][0]
                e = jnp.where(j < nthis, p >> 9, 0)
                pltpu.async_copy(cflat.at[pl.ds(e * h, h)],
                                 rows.at[pl.ds(j * h, h)], sem)
                return c
            lax.fori_loop(0, rb, f, 0)

        def rmw(wlbuf, rows, nthis):
            def r(j, c):
                p = wlbuf[pl.ds(j * 16, 16)][0]
                n_i = jnp.where(j < nthis, p & 511, _NPW)
                for kk in range(nk):
                    t = table[n_i, pl.ds(kk * 16, 16)]
                    rr = rows[pl.ds(j * h + kk * 16, 16)]
                    table[n_i, pl.ds(kk * 16, 16)] = jnp.maximum(t, rr)
                return c
            lax.fori_loop(0, rb, r, 0)

        def scanner(v, merged):
            pltpu.sync_copy(rcnt.at[pl.ds((v * _NW + w) * 16, 16)], stg)
            tc = stg[pl.ds(0, 16)][0]
            nsub2 = (tc + 2 * rb - 1) // (2 * rb)
            rbase = (v * _NW + w) * _RCAP

            def sub2(s2, c):
                for b in range(2):
                    base = (s2 * 2 + b) * rb
                    @pl.when(s2 > 0)
                    def _():
                        pltpu.make_async_copy(
                            routed.at[pl.ds(0, rb * 16)], wlbufs[b],
                            sem2).wait()
                    pltpu.sync_copy(
                        routed.at[pl.ds((rbase + base) * 16, rb * 16)],
                        wlbufs[b])
                    pltpu.async_copy(
                        wlbufs[b],
                        wl_hbm.at[pl.ds((w * _WLREG + merged + base) * 16,
                                        rb * 16)], sem2)
                    fire(wlbufs[b], rowss[b], sems[b], tc - base)
                for b in range(2):
                    base = (s2 * 2 + b) * rb
                    pltpu.make_async_copy(cflat.at[pl.ds(0, rb * h)],
                                          rowss[b], sems[b]).wait()
                    rmw(wlbufs[b], rowss[b], tc - base)
                return c

            lax.fori_loop(0, nsub2, sub2, 0)

            @pl.when(nsub2 > 0)
            def _():
                for b in range(2):
                    pltpu.make_async_copy(routed.at[pl.ds(0, rb * 16)],
                                          wlbufs[b], sem2).wait()
            return merged + tc

        merged = lax.fori_loop(0, _NW, scanner, 0)

        stg[pl.ds(0, 16)] = zeros_i + merged
        pltpu.sync_copy(stg, wcnt_hbm.at[pl.ds(w * 16, 16)])

        _table_finalize(table, bb_v, out_hbm, lo, nk)

    return k


def _edge_conv(x, dst, src, Wa, ba, Wb, bb, hin, h, wl_state):
    wd = Wa[:hin] - Wa[hin:]
    ws = Wa[hin:]
    hp = max(h, 128)
    if hp > h:
        wd = jnp.pad(wd, ((0, 0), (0, hp - h)))
        ws = jnp.pad(ws, ((0, 0), (0, hp - h)))
        ba = jnp.pad(ba, (0, hp - h))
        Wb = jnp.pad(Wb, ((0, hp - h), (0, hp - h)))
        bb = jnp.pad(bb, (0, hp - h))
    if hin < 8:
        pad = 8 - hin
        x = jnp.pad(x, ((0, 0), (0, pad)))
        wd = jnp.pad(wd, ((0, pad), (0, 0)))
        ws = jnp.pad(ws, ((0, pad), (0, 0)))
    p, q = _project(x, wd, ws, ba)
    pa, pb = _make_gather(hp)(p, q, dst, src)
    c = _edge_mlp(pa, pb, Wb)
    cflat = c.reshape(-1)
    if isinstance(wl_state, tuple) and len(wl_state) == 2 and \
            wl_state[0] is not None and wl_state[1] is None:
        routed, rcnt = wl_state[0]
        xn, wl_hbm, wcnt = _make_segmax_merge(hp)(cflat, routed, rcnt, bb)
        wl_state = (wl_hbm, wcnt)
    else:
        xn = _make_segmax_apply(hp)(cflat, wl_state[0], wl_state[1], bb)
    return xn[:_N_NODES, :h], wl_state


def kernel(x, edge_index, batch, W1a, b1a, W1b, b1b, W2a, b2a, W2b, b2b,
           W3a, b3a, W3b, b3b, Wfc, bfc):
    src = edge_index[0]
    dst = edge_index[1]
    routed, rcnt = _make_route()(dst)
    x1, wls = _edge_conv(x, dst, src, W1a, b1a, W1b, b1b, 3, 64,
                         ((routed, rcnt), None))
    x2, wls = _edge_conv(x1, dst, src, W2a, b2a, W2b, b2b, 64, 128, wls)
    x3, wls = _edge_conv(x2, dst, src, W3a, b3a, W3b, b3b, 128, 256, wls)
    return _pool(x3, batch, Wfc, bfc)


# speedup vs baseline: 1.2025x; 1.2025x over previous
import functools

import jax
import jax.numpy as jnp
from jax import lax
from jax.experimental import pallas as pl
from jax.experimental.pallas import tpu as pltpu
from jax.experimental.pallas import tpu_sc as plsc

_N_NODES = 10000
_N_EDGES = 320000
_N_GRAPHS = 64
_NW = 32
_NPW = 320
_NPAD = _NW * _NPW
_WLREG = _N_EDGES + 64

_MESH = plsc.VectorSubcoreMesh(
    core_axis_name="c", subcore_axis_name="s", num_cores=2, num_subcores=16)


def _wid():
    return lax.axis_index("s") * 2 + lax.axis_index("c")


def _proj_body(x_ref, wd_ref, ws_ref, ba_ref, p_ref, q_ref):
    x = x_ref[...]
    p_ref[...] = (jnp.dot(x, wd_ref[...], preferred_element_type=jnp.float32)
                  + ba_ref[...])
    q_ref[...] = jnp.dot(x, ws_ref[...], preferred_element_type=jnp.float32)


def _project(x, wd, ws, ba):
    n, hin = x.shape
    h = wd.shape[1]
    bm = 2000
    return pl.pallas_call(
        _proj_body,
        grid=(n // bm,),
        in_specs=[
            pl.BlockSpec((bm, hin), lambda i: (i, 0)),
            pl.BlockSpec((hin, h), lambda i: (0, 0)),
            pl.BlockSpec((hin, h), lambda i: (0, 0)),
            pl.BlockSpec((1, h), lambda i: (0, 0)),
        ],
        out_specs=[
            pl.BlockSpec((bm, h), lambda i: (i, 0)),
            pl.BlockSpec((bm, h), lambda i: (i, 0)),
        ],
        out_shape=[
            jax.ShapeDtypeStruct((n, h), jnp.float32),
            jax.ShapeDtypeStruct((n, h), jnp.float32),
        ],
    )(x, wd, ws, ba.reshape(1, h))


def _edge_mlp_body(a_ref, b_ref, wb_ref, c_ref):
    h = jnp.maximum(a_ref[...] + b_ref[...], 0.0)
    c_ref[...] = jnp.dot(h, wb_ref[...], preferred_element_type=jnp.float32)


def _edge_mlp(pa, pb, wb):
    e, h = pa.shape
    bm = 2000
    return pl.pallas_call(
        _edge_mlp_body,
        grid=(e // bm,),
        in_specs=[
            pl.BlockSpec((bm, h), lambda i: (i, 0)),
            pl.BlockSpec((bm, h), lambda i: (i, 0)),
            pl.BlockSpec((h, h), lambda i: (0, 0)),
        ],
        out_specs=pl.BlockSpec((bm, h), lambda i: (i, 0)),
        out_shape=jax.ShapeDtypeStruct((e, h), jnp.float32),
    )(pa, pb, wb)


def _pool_body(x_ref, batch_ref, wfc_ref, bfc_ref, out_ref, pooled_ref):
    xb = x_ref[...]
    bvec = batch_ref[...]

    def body(g, c):
        mg = jnp.max(jnp.where(bvec == g, xb, -jnp.inf), axis=0,
                     keepdims=True)
        pooled_ref[pl.ds(g, 1), :] = jnp.where(mg == -jnp.inf, 0.0, mg)
        return c

    lax.fori_loop(0, _N_GRAPHS, body, 0)
    out_ref[...] = (jnp.dot(pooled_ref[...], wfc_ref[...],
                            preferred_element_type=jnp.float32) + bfc_ref[...])


def _pool(x3, batch, wfc, bfc):
    n, h = x3.shape
    hout = wfc.shape[1]
    return pl.pallas_call(
        _pool_body,
        out_shape=jax.ShapeDtypeStruct((_N_GRAPHS, hout), jnp.float32),
        scratch_shapes=[pltpu.VMEM((_N_GRAPHS, h), jnp.float32)],
    )(x3, batch.reshape(n, 1), wfc, bfc.reshape(1, hout))


@functools.cache
def _make_gather(h):
    ch = 128
    epw = _N_EDGES // _NW
    nfull = epw // ch
    rem = epw - nfull * ch

    @functools.partial(
        pl.kernel,
        out_type=[
            jax.ShapeDtypeStruct((_N_EDGES, h), jnp.float32),
            jax.ShapeDtypeStruct((_N_EDGES, h), jnp.float32),
        ],
        mesh=_MESH,
        scratch_types=[
            pltpu.VMEM((ch,), jnp.int32),
            pltpu.VMEM((ch,), jnp.int32),
            pltpu.VMEM((ch, h), jnp.float32),
            pltpu.VMEM((ch, h), jnp.float32),
            pltpu.SemaphoreType.DMA,
            pltpu.SemaphoreType.DMA,
        ],
    )
    def k(p_hbm, q_hbm, dst_hbm, src_hbm, outa, outb,
          idx_d, idx_s, buf_a, buf_b, sem_a, sem_b):
        base = _wid() * epw

        def do_chunk(off, n):
            pltpu.sync_copy(dst_hbm.at[pl.ds(off, n)], idx_d.at[pl.ds(0, n)])
            pltpu.sync_copy(src_hbm.at[pl.ds(off, n)], idx_s.at[pl.ds(0, n)])
            cpa = pltpu.async_copy(p_hbm.at[idx_d.at[pl.ds(0, n)]],
                                   buf_a.at[pl.ds(0, n)], sem_a)
            cpb = pltpu.async_copy(q_hbm.at[idx_s.at[pl.ds(0, n)]],
                                   buf_b.at[pl.ds(0, n)], sem_b)
            cpa.wait()
            cpb.wait()
            pltpu.sync_copy(buf_a.at[pl.ds(0, n)], outa.at[pl.ds(off, n)])
            pltpu.sync_copy(buf_b.at[pl.ds(0, n)], outb.at[pl.ds(off, n)])

        def body(i, carry):
            do_chunk(base + i * ch, ch)
            return carry

        lax.fori_loop(0, nfull, body, 0)
        do_chunk(base + nfull * ch, rem)

    return k


def _fire_drain_rmw(cflat, sem, wl, wlbase, rows, table, nthis, rb, h, nk):
    def fire(j, c):
        p = wl[pl.ds((wlbase + j) * 16, 16)][0]
        e = jnp.where(j < nthis, p >> 9, 0)
        pltpu.async_copy(cflat.at[pl.ds(e * h, h)],
                         rows.at[pl.ds(j * h, h)], sem)
        return c

    lax.fori_loop(0, rb, fire, 0)
    pltpu.make_async_copy(cflat.at[pl.ds(0, rb * h)], rows, sem).wait()

    def rmw(j, c):
        p = wl[pl.ds((wlbase + j) * 16, 16)][0]
        n_i = jnp.where(j < nthis, p & 511, _NPW)
        for kk in range(nk):
            t = table[n_i, pl.ds(kk * 16, 16)]
            r = rows[pl.ds(j * h + kk * 16, 16)]
            table[n_i, pl.ds(kk * 16, 16)] = jnp.maximum(t, r)
        return c

    lax.fori_loop(0, rb, rmw, 0)


def _table_init(table, nk):
    neg_inf = jnp.full((16,), -jnp.inf, jnp.float32)

    def init_row(j, c):
        for kk in range(nk):
            table[j, pl.ds(kk * 16, 16)] = neg_inf
        return c

    lax.fori_loop(0, _NPW + 1, init_row, 0)


def _table_finalize(table, bb_v, out_hbm, lo, nk):
    neg_inf = jnp.full((16,), -jnp.inf, jnp.float32)

    def fin_row(j, c):
        for kk in range(nk):
            t = table[j, pl.ds(kk * 16, 16)]
            b = bb_v[pl.ds(kk * 16, 16)]
            table[j, pl.ds(kk * 16, 16)] = jnp.where(t == neg_inf, 0.0, t + b)
        return c

    lax.fori_loop(0, _NPW, fin_row, 0)
    pltpu.sync_copy(table.at[pl.ds(0, _NPW)], out_hbm.at[pl.ds(lo, _NPW)])


@functools.cache
def _make_segmax_build(h):
    ch2 = 256
    ngrp = ch2 // 16
    wlcap = 768
    flush_at = 512
    rb = 64
    nk = h // 16

    @functools.partial(
        pl.kernel,
        out_type=[
            jax.ShapeDtypeStruct((_NPAD, h), jnp.float32),
            jax.ShapeDtypeStruct((_NW * _WLREG * 16,), jnp.int32),
            jax.ShapeDtypeStruct((_NW * 16,), jnp.int32),
        ],
        mesh=_MESH,
        scratch_types=[
            pltpu.VMEM((_NPW + 1, h), jnp.float32),
            pltpu.VMEM((ch2,), jnp.int32),
            pltpu.VMEM(((wlcap + 1) * 16,), jnp.int32),
            pltpu.VMEM((rb * h,), jnp.float32),
            pltpu.VMEM((h,), jnp.float32),
            pltpu.VMEM((16,), jnp.int32),
            pltpu.SemaphoreType.DMA,
            pltpu.SemaphoreType.DMA,
        ],
    )
    def k(cflat, dst_hbm, bb_hbm, out_hbm, wl_hbm, wcnt_hbm,
          table, dvec, wl, rows, bb_v, stg, sem, sem2):
        w = _wid()
        lo = w * _NPW
        zeros_i = jnp.zeros((16,), jnp.int32)

        pltpu.sync_copy(bb_hbm, bb_v)
        _table_init(table, nk)

        def flush(cnt, cursor):
            nsub = (cnt + rb - 1) // rb

            def sub(s, c):
                base = s * rb
                nthis = cnt - base
                pltpu.async_copy(
                    wl.at[pl.ds(base * 16, rb * 16)],
                    wl_hbm.at[pl.ds((w * _WLREG + cursor + base) * 16,
                                    rb * 16)],
                    sem2)
                _fire_drain_rmw(cflat, sem, wl, base, rows, table, nthis,
                                rb, h, nk)
                pltpu.make_async_copy(
                    cflat.at[pl.ds(0, rb * 16)],
                    wl.at[pl.ds(0, rb * 16)], sem2).wait()
                return c

            lax.fori_loop(0, nsub, sub, 0)
            return cursor + cnt

        def chunk(ci, carry):
            cnt0, cursor0 = carry
            coff = ci * ch2
            pltpu.sync_copy(dst_hbm.at[pl.ds(coff, ch2)], dvec)

            def grp(g, cnt_in):
                d = dvec[pl.ds(g * 16, 16)]
                dl = d - lo
                cnt = cnt_in
                for lane in range(16):
                    dl_l = dl[lane]
                    ok = (dl_l >= 0) & (dl_l < _NPW)
                    e = coff + g * 16 + lane
                    val = zeros_i + (e * 512 + dl_l)
                    slot = jnp.where(ok, cnt, wlcap)
                    wl[pl.ds(slot * 16, 16)] = val
                    cnt = cnt + ok.astype(jnp.int32)
                return cnt

            cnt1 = lax.fori_loop(0, ngrp, grp, cnt0)
            return lax.cond(cnt1 >= flush_at,
                            lambda a: (0, flush(a[0], a[1])),
                            lambda a: a, (cnt1, cursor0))

        cnt_f, cursor_f = lax.fori_loop(0, _N_EDGES // ch2, chunk, (0, 0))
        cursor_f = lax.cond(cnt_f > 0,
                            lambda a: flush(a[0], a[1]),
                            lambda a: a[1], (cnt_f, cursor_f))

        stg[pl.ds(0, 16)] = zeros_i + cursor_f
        pltpu.sync_copy(stg, wcnt_hbm.at[pl.ds(w * 16, 16)])

        _table_finalize(table, bb_v, out_hbm, lo, nk)

    return k


@functools.cache
def _make_segmax_apply(h):
    rb = 64
    nk = h // 16

    @functools.partial(
        pl.kernel,
        out_type=jax.ShapeDtypeStruct((_NPAD, h), jnp.float32),
        mesh=_MESH,
        scratch_types=[
            pltpu.VMEM((_NPW + 1, h), jnp.float32),
            pltpu.VMEM((rb * 16,), jnp.int32),
            pltpu.VMEM((rb * 16,), jnp.int32),
            pltpu.VMEM((rb * h,), jnp.float32),
            pltpu.VMEM((rb * h,), jnp.float32),
            pltpu.VMEM((h,), jnp.float32),
            pltpu.VMEM((16,), jnp.int32),
            pltpu.SemaphoreType.DMA,
            pltpu.SemaphoreType.DMA,
        ],
    )
    def k(cflat, wl_hbm, wcnt_hbm, bb_hbm, out_hbm,
          table, wlbuf0, wlbuf1, rows0, rows1, bb_v, stg, sem0, sem1):
        w = _wid()
        lo = w * _NPW
        wlbufs = (wlbuf0, wlbuf1)
        rowss = (rows0, rows1)
        sems = (sem0, sem1)

        pltpu.sync_copy(bb_hbm, bb_v)
        _table_init(table, nk)

        pltpu.sync_copy(wcnt_hbm.at[pl.ds(w * 16, 16)], stg)
        n = stg[pl.ds(0, 16)][0]
        nsub2 = (n + 2 * rb - 1) // (2 * rb)

        def fire(wlbuf, rows, sem, nthis):
            def f(j, c):
                p = wlbuf[pl.ds(j * 16, 16)][0]
                e = jnp.where(j < nthis, p >> 9, 0)
                pltpu.async_copy(cflat.at[pl.ds(e * h, h)],
                                 rows.at[pl.ds(j * h, h)], sem)
                return c
            lax.fori_loop(0, rb, f, 0)

        def rmw(wlbuf, rows, nthis):
            def r(j, c):
                p = wlbuf[pl.ds(j * 16, 16)][0]
                n_i = jnp.where(j < nthis, p & 511, _NPW)
                for kk in range(nk):
                    t = table[n_i, pl.ds(kk * 16, 16)]
                    rr = rows[pl.ds(j * h + kk * 16, 16)]
                    table[n_i, pl.ds(kk * 16, 16)] = jnp.maximum(t, rr)
                return c
            lax.fori_loop(0, rb, r, 0)

        def sub2(s2, c):
            for b in range(2):
                base = (s2 * 2 + b) * rb
                pltpu.sync_copy(
                    wl_hbm.at[pl.ds((w * _WLREG + base) * 16, rb * 16)],
                    wlbufs[b])
                fire(wlbufs[b], rowss[b], sems[b], n - base)
            for b in range(2):
                base = (s2 * 2 + b) * rb
                pltpu.make_async_copy(cflat.at[pl.ds(0, rb * h)],
                                      rowss[b], sems[b]).wait()
                rmw(wlbufs[b], rowss[b], n - base)
            return c

        lax.fori_loop(0, nsub2, sub2, 0)

        _table_finalize(table, bb_v, out_hbm, lo, nk)

    return k


_RBUF = 32
_EPW = _N_EDGES // _NW
_RCAP = _EPW + _RBUF


@functools.cache
def _make_route():
    ch = 400
    nch = _EPW // ch

    @functools.partial(
        pl.kernel,
        out_type=[
            jax.ShapeDtypeStruct((_NW * _NW * _RCAP * 16,), jnp.int32),
            jax.ShapeDtypeStruct((_NW * _NW * 16,), jnp.int32),
        ],
        mesh=_MESH,
        scratch_types=[
            pltpu.VMEM((ch,), jnp.int32),
            pltpu.VMEM((_NW * _RBUF * 16,), jnp.int32),
            pltpu.VMEM((_NW * 16,), jnp.int32),
            pltpu.VMEM((16,), jnp.int32),
            pltpu.SemaphoreType.DMA,
        ],
    )
    def k(dst_hbm, routed, rcnt, dvec, stage, cnt, stg, sem):
        v = _wid()
        base_e = v * _EPW
        zeros_i = jnp.zeros((16,), jnp.int32)

        def zinit(j, c):
            cnt[pl.ds(j * 16, 16)] = zeros_i
            return c
        lax.fori_loop(0, _NW, zinit, 0)

        def zinit2(j, c):
            stage[pl.ds(j * 16, 16)] = zeros_i
            return c
        lax.fori_loop(0, _NW * _RBUF, zinit2, 0)

        def chunk(ci, c):
            coff = ci * ch
            pltpu.sync_copy(dst_hbm.at[pl.ds(base_e + coff, ch)], dvec)

            def grp(g, c2):
                d = dvec[pl.ds(g * 16, 16)]
                for lane in range(16):
                    d_l = d[lane]
                    ow = ((d_l >> 6) * 6554) >> 15
                    dl = d_l - ow * 320
                    e = base_e + coff + g * 16 + lane
                    tc = cnt[pl.ds(ow * 16, 16)][0]
                    stage[pl.ds((ow * _RBUF + (tc & (_RBUF - 1))) * 16,
                                16)] = zeros_i + (e * 512 + dl)
                    cnt[pl.ds(ow * 16, 16)] = zeros_i + (tc + 1)

                    @pl.when((tc & (_RBUF - 1)) == _RBUF - 1)
                    def _():
                        pltpu.sync_copy(
                            stage.at[pl.ds(ow * _RBUF * 16, _RBUF * 16)],
                            routed.at[pl.ds(
                                ((v * _NW + ow) * _RCAP + tc - (_RBUF - 1))
                                * 16, _RBUF * 16)])
                return c2

            lax.fori_loop(0, ch // 16, grp, 0)
            return c

        lax.fori_loop(0, nch, chunk, 0)

        def fin(ow, c):
            tc = cnt[pl.ds(ow * 16, 16)][0]

            @pl.when((tc & (_RBUF - 1)) != 0)
            def _():
                pltpu.sync_copy(
                    stage.at[pl.ds(ow * _RBUF * 16, _RBUF * 16)],
                    routed.at[pl.ds(
                        ((v * _NW + ow) * _RCAP + (tc & ~(_RBUF - 1))) * 16,
                        _RBUF * 16)])

            stg[pl.ds(0, 16)] = zeros_i + tc
            pltpu.sync_copy(stg, rcnt.at[pl.ds((v * _NW + ow) * 16, 16)])
            return c

        lax.fori_loop(0, _NW, fin, 0)

    return k


@functools.cache
def _make_segmax_merge(h):
    rb = 64
    nk = h // 16

    @functools.partial(
        pl.kernel,
        out_type=[
            jax.ShapeDtypeStruct((_NPAD, h), jnp.float32),
            jax.ShapeDtypeStruct((_NW * _WLREG * 16,), jnp.int32),
            jax.ShapeDtypeStruct((_NW * 16,), jnp.int32),
        ],
        mesh=_MESH,
        scratch_types=[
            pltpu.VMEM((_NPW + 1, h), jnp.float32),
            pltpu.VMEM((rb * 16,), jnp.int32),
            pltpu.VMEM((rb * h,), jnp.float32),
            pltpu.VMEM((h,), jnp.float32),
            pltpu.VMEM((16,), jnp.int32),
            pltpu.SemaphoreType.DMA,
            pltpu.SemaphoreType.DMA,
        ],
    )
    def k(cflat, routed, rcnt, bb_hbm, out_hbm, wl_hbm, wcnt_hbm,
          table, wlbuf, rows, bb_v, stg, sem, sem2):
        w = _wid()
        lo = w * _NPW
        zeros_i = jnp.zeros((16,), jnp.int32)

        pltpu.sync_copy(bb_hbm, bb_v)
        _table_init(table, nk)

        def scanner(v, merged):
            pltpu.sync_copy(rcnt.at[pl.ds((v * _NW + w) * 16, 16)], stg)
            tc = stg[pl.ds(0, 16)][0]
            nsub = (tc + rb - 1) // rb
            rbase = (v * _NW + w) * _RCAP

            def sub(si, c):
                base = si * rb
                pltpu.sync_copy(
                    routed.at[pl.ds((rbase + base) * 16, rb * 16)], wlbuf)
                cpw = pltpu.async_copy(
                    wlbuf,
                    wl_hbm.at[pl.ds((w * _WLREG + merged + base) * 16,
                                    rb * 16)], sem2)
                _fire_drain_rmw(cflat, sem, wlbuf, 0, rows, table,
                                tc - base, rb, h, nk)
                cpw.wait()
                return c

            lax.fori_loop(0, nsub, sub, 0)
            return merged + tc

        merged = lax.fori_loop(0, _NW, scanner, 0)

        stg[pl.ds(0, 16)] = zeros_i + merged
        pltpu.sync_copy(stg, wcnt_hbm.at[pl.ds(w * 16, 16)])

        _table_finalize(table, bb_v, out_hbm, lo, nk)

    return k


def _edge_conv(x, dst, src, Wa, ba, Wb, bb, hin, h, wl_state):
    wd = Wa[:hin] - Wa[hin:]
    ws = Wa[hin:]
    hp = max(h, 128)
    if hp > h:
        wd = jnp.pad(wd, ((0, 0), (0, hp - h)))
        ws = jnp.pad(ws, ((0, 0), (0, hp - h)))
        ba = jnp.pad(ba, (0, hp - h))
        Wb = jnp.pad(Wb, ((0, hp - h), (0, hp - h)))
        bb = jnp.pad(bb, (0, hp - h))
    if hin < 8:
        pad = 8 - hin
        x = jnp.pad(x, ((0, 0), (0, pad)))
        wd = jnp.pad(wd, ((0, pad), (0, 0)))
        ws = jnp.pad(ws, ((0, pad), (0, 0)))
    p, q = _project(x, wd, ws, ba)
    pa, pb = _make_gather(hp)(p, q, dst, src)
    c = _edge_mlp(pa, pb, Wb)
    cflat = c.reshape(-1)
    if isinstance(wl_state, tuple) and len(wl_state) == 2 and \
            wl_state[0] is not None and wl_state[1] is None:
        routed, rcnt = wl_state[0]
        xn, wl_hbm, wcnt = _make_segmax_merge(hp)(cflat, routed, rcnt, bb)
        wl_state = (wl_hbm, wcnt)
    else:
        xn = _make_segmax_apply(hp)(cflat, wl_state[0], wl_state[1], bb)
    return xn[:_N_NODES, :h], wl_state


def kernel(x, edge_index, batch, W1a, b1a, W1b, b1b, W2a, b2a, W2b, b2b,
           W3a, b3a, W3b, b3b, Wfc, bfc):
    src = edge_index[0]
    dst = edge_index[1]
    routed, rcnt = _make_route()(dst)
    x1, wls = _edge_conv(x, dst, src, W1a, b1a, W1b, b1b, 3, 64,
                         ((routed, rcnt), None))
    x2, wls = _edge_conv(x1, dst, src, W2a, b2a, W2b, b2b, 64, 128, wls)
    x3, wls = _edge_conv(x2, dst, src, W3a, b3a, W3b, b3b, 128, 256, wls)
    return _pool(x3, batch, Wfc, bfc)
